# Initial kernel scaffold; baseline (speedup 1.0000x reference)
import jax
import jax.numpy as jnp
from jax.experimental import pallas as pl


def kernel(edge_emb, W1, W2, Hin_idx, Hin_val, Hout_idx, Hout_val, edge_attr):
    raise NotImplementedError("write your pallas kernel here")


# speedup vs baseline: 4.9033x; 4.9033x over previous
import functools

import jax
import jax.numpy as jnp
from jax import lax
from jax.experimental import pallas as pl
from jax.experimental.pallas import tpu as pltpu
from jax.experimental.pallas import tpu_sc as plsc

N = 50000
E = 800000
D = 64
R = 10
HALF = 32

NS = 16
SUB = 80
NSUB = 5
C = SUB * NSUB
EPT = E // NS
NCH = EPT // C
RPT = N // NS
ZR = 125


def _spmm_body(x2, rin, cin, vin, rout, cout, vout, nin2, nout2,
               acc, colv, gidx, valv, rowv1, rowv, G, zbuf, sem):
    core = lax.axis_index("c")
    sid = lax.axis_index("s")

    def _zb(r, _):
        z16 = jnp.zeros((16,), jnp.float32)
        zbuf[r, pl.ds(0, 16)] = z16
        zbuf[r, pl.ds(16, 16)] = z16
        return 0
    lax.fori_loop(0, ZR, _zb, 0)

    def run_pass(rows_hbm, cols_hbm, vals_hbm, out_hbm):
        for z in range(RPT // ZR):
            pltpu.sync_copy(zbuf, acc.at[pl.ds(sid * RPT + z * ZR, ZR)])
        plsc.subcore_barrier()

        def chunk(i, _):
            ebase = sid * EPT + i * C
            pltpu.sync_copy(cols_hbm.at[pl.ds(ebase, C)], colv)
            pltpu.sync_copy(vals_hbm.at[pl.ds(ebase, C)], valv)
            pltpu.sync_copy(rows_hbm.at[pl.ds(ebase, C)], rowv1)

            for j in range(NSUB):
                def _gi(g, _):
                    o = j * SUB + g * 16
                    c16 = colv[pl.ds(o, 16)]
                    gidx[pl.ds(o, 16)] = c16 + c16 + core
                    rowv[j, pl.ds(g * 16, 16)] = rowv1[pl.ds(o, 16)]
                    return 0
                lax.fori_loop(0, SUB // 16, _gi, 0)

            cps = [pltpu.async_copy(x2.at[gidx.at[pl.ds(j * SUB, SUB)]],
                                    G.at[pl.ds(j * SUB, SUB)], sem)
                   for j in range(NSUB)]
            for cp in cps:
                cp.wait()

            def _sc(fg, _):
                o = fg * 16
                v16 = valv[pl.ds(o, 16)]
                for e in range(16):
                    r = o + e
                    b = v16[e]
                    G[r, pl.ds(0, 16)] = G[r, pl.ds(0, 16)] * b
                    G[r, pl.ds(16, 16)] = G[r, pl.ds(16, 16)] * b
                return 0
            lax.fori_loop(0, C // 16, _sc, 0)

            for j in range(NSUB):
                pltpu.sync_copy(G.at[pl.ds(j * SUB, SUB)],
                                acc.at[rowv.at[j]], add=True)
            return 0

        lax.fori_loop(0, NCH, chunk, 0)
        plsc.subcore_barrier()
        pltpu.sync_copy(acc.at[pl.ds(sid * RPT, RPT)],
                        out_hbm.at[core, pl.ds(sid * RPT, RPT)])
        plsc.subcore_barrier()

    run_pass(rin, cin, vin, nin2)
    run_pass(rout, cout, vout, nout2)


@jax.jit
def _spmm2(x2, rin, cin, vin, rout, cout, vout):
    mesh = plsc.VectorSubcoreMesh(core_axis_name="c", subcore_axis_name="s")
    f = pl.kernel(
        _spmm_body,
        out_type=(jax.ShapeDtypeStruct((2, N, HALF), jnp.float32),
                  jax.ShapeDtypeStruct((2, N, HALF), jnp.float32)),
        mesh=mesh,
        scratch_types=(
            pltpu.VMEM_SHARED((N, HALF), jnp.float32),
            pltpu.VMEM((C,), jnp.int32),
            pltpu.VMEM((C,), jnp.int32),
            pltpu.VMEM((C,), jnp.float32),
            pltpu.VMEM((C,), jnp.int32),
            pltpu.VMEM((NSUB, SUB), jnp.int32),
            pltpu.VMEM((C, HALF), jnp.float32),
            pltpu.VMEM((ZR, HALF), jnp.float32),
            pltpu.SemaphoreType.DMA,
        ),
        compiler_params=pltpu.CompilerParams(use_tc_tiling_on_sc=False),
    )
    return f(x2, rin, cin, vin, rout, cout, vout)


BN = 2000


def _layer_body(ego_ref, nin_ref, nout_ref, w_ref, ego_o_ref, emb_o_ref):
    ego = ego_ref[...]
    w = w_ref[...]
    acc = jnp.dot(ego, w[0:64], preferred_element_type=jnp.float32)
    acc = acc + jnp.dot(nin_ref[0], w[64:96], preferred_element_type=jnp.float32)
    acc = acc + jnp.dot(nin_ref[1], w[96:128], preferred_element_type=jnp.float32)
    acc = acc + jnp.dot(nout_ref[0], w[128:160], preferred_element_type=jnp.float32)
    acc = acc + jnp.dot(nout_ref[1], w[160:192], preferred_element_type=jnp.float32)
    ego_n = jnp.where(acc >= 0, acc, 0.01 * acc)
    ego_o_ref[...] = ego_n
    nrm = jnp.sqrt(jnp.sum(ego_n * ego_n, axis=1, keepdims=True))
    emb_o_ref[...] = ego_n / jnp.maximum(nrm, 1e-12)


@jax.jit
def _layer(ego, nin2, nout2, w):
    return pl.pallas_call(
        _layer_body,
        grid=(N // BN,),
        in_specs=[
            pl.BlockSpec((BN, D), lambda i: (i, 0)),
            pl.BlockSpec((2, BN, HALF), lambda i: (0, i, 0)),
            pl.BlockSpec((2, BN, HALF), lambda i: (0, i, 0)),
            pl.BlockSpec((3 * D, D), lambda i: (0, 0)),
        ],
        out_specs=[
            pl.BlockSpec((BN, D), lambda i: (i, 0)),
            pl.BlockSpec((BN, D), lambda i: (i, 0)),
        ],
        out_shape=[
            jax.ShapeDtypeStruct((N, D), jnp.float32),
            jax.ShapeDtypeStruct((N, D), jnp.float32),
        ],
    )(ego, nin2, nout2, w)


def _kg_body(attr_ref, e0_ref, e1_ref, e2_ref, out_ref):
    i = pl.program_id(0)
    a = attr_ref[...].reshape(BN, 1)
    lbl = lax.broadcasted_iota(jnp.int32, (BN, R), 1).astype(jnp.float32)
    oh = (a == lbl).astype(jnp.float32)
    dn = (((0,), (0,)), ((), ()))
    r0 = lax.dot_general(oh, e0_ref[...], dn, preferred_element_type=jnp.float32)
    r1 = lax.dot_general(oh, e1_ref[...], dn, preferred_element_type=jnp.float32)
    r2 = lax.dot_general(oh, e2_ref[...], dn, preferred_element_type=jnp.float32)
    contrib = jnp.concatenate([r0, r1, r2], axis=1)

    @pl.when(i == 0)
    def _():
        out_ref[...] = jnp.zeros_like(out_ref)

    out_ref[...] += contrib


@jax.jit
def _kg(attr_f, e0, e1, e2):
    return pl.pallas_call(
        _kg_body,
        grid=(N // BN,),
        in_specs=[
            pl.BlockSpec((1, 1, BN), lambda i: (i, 0, 0)),
            pl.BlockSpec((BN, D), lambda i: (i, 0)),
            pl.BlockSpec((BN, D), lambda i: (i, 0)),
            pl.BlockSpec((BN, D), lambda i: (i, 0)),
        ],
        out_specs=pl.BlockSpec((R, 3 * D), lambda i: (0, 0)),
        out_shape=jax.ShapeDtypeStruct((R, 3 * D), jnp.float32),
    )(attr_f, e0, e1, e2)


def kernel(edge_emb, W1, W2, Hin_idx, Hin_val, Hout_idx, Hout_val, edge_attr):
    rin = Hin_idx[0]
    cin = Hin_idx[1]
    rout = Hout_idx[0]
    cout = Hout_idx[1]

    ego0 = edge_emb
    nin2, nout2 = _spmm2(ego0.reshape(2 * N, HALF), rin, cin, Hin_val,
                         rout, cout, Hout_val)
    ego1, emb1 = _layer(ego0, nin2, nout2, W1)
    nin2b, nout2b = _spmm2(ego1.reshape(2 * N, HALF), rin, cin, Hin_val,
                           rout, cout, Hout_val)
    _, emb2 = _layer(ego1, nin2b, nout2b, W2)

    attr_f = edge_attr.astype(jnp.float32).reshape(N // BN, 1, BN)
    return _kg(attr_f, ego0, emb1, emb2)


# speedup vs baseline: 8.7057x; 1.7755x over previous
import functools

import jax
import jax.numpy as jnp
from jax import lax
from jax.experimental import pallas as pl
from jax.experimental.pallas import tpu as pltpu
from jax.experimental.pallas import tpu_sc as plsc

N = 50000
E = 800000
D = 64
R = 10
HALF = 32

NS = 16
SUB = 80
NSUB = 5
C = SUB * NSUB
EPT = E // NS
NCH = EPT // C
RPT = N // NS
ZR = 25


def _spmm_body(x2, gin, rin, vin, gout, rout, vout, nin2, nout2,
               acc, gv, vv, rv, G, zbuf, sem_in, sem_g, sem_s):
    core = lax.axis_index("c")
    sid = lax.axis_index("s")

    def _zb(r, _):
        z16 = jnp.zeros((16,), jnp.float32)
        zbuf[r, pl.ds(0, 16)] = z16
        zbuf[r, pl.ds(16, 16)] = z16
        return 0
    lax.fori_loop(0, ZR, _zb, 0)

    def run_pass(g_hbm, rows_hbm, vals_hbm, out_hbm):
        def _zz(zg, _):
            cps = [pltpu.async_copy(
                zbuf, acc.at[pl.ds(sid * RPT + (zg * 5 + u) * ZR, ZR)], sem_s)
                for u in range(5)]
            for cp in cps:
                cp.wait()
            return 0
        lax.fori_loop(0, RPT // ZR // 5, _zz, 0)
        plsc.subcore_barrier()

        def issue_in(k, b):
            ebase = sid * EPT + k * C
            pltpu.async_copy(g_hbm.at[core, pl.ds(ebase, C)], gv.at[b], sem_in)
            pltpu.async_copy(vals_hbm.at[pl.ds(ebase, C)], vv.at[b], sem_in)
            for j in range(NSUB):
                pltpu.async_copy(rows_hbm.at[pl.ds(ebase + j * SUB, SUB)],
                                 rv.at[b, j], sem_in)

        def wait_in(b):
            pltpu.make_async_copy(g_hbm.at[core, pl.ds(0, C)],
                                  gv.at[b], sem_in).wait()
            pltpu.make_async_copy(vals_hbm.at[pl.ds(0, C)],
                                  vv.at[b], sem_in).wait()
            for j in range(NSUB):
                pltpu.make_async_copy(rows_hbm.at[pl.ds(0, SUB)],
                                      rv.at[b, j], sem_in).wait()

        def issue_gather(b):
            for j in range(NSUB):
                pltpu.async_copy(x2.at[gv.at[b, pl.ds(j * SUB, SUB)]],
                                 G.at[b, pl.ds(j * SUB, SUB)], sem_g)

        def wait_gather(b):
            for j in range(NSUB):
                pltpu.make_async_copy(
                    x2.at[gv.at[b, pl.ds(j * SUB, SUB)]],
                    G.at[b, pl.ds(j * SUB, SUB)], sem_g).wait()

        def scale(b):
            def _sc(fg, _):
                o = fg * 16
                v16 = vv[b, pl.ds(o, 16)]
                for e in range(16):
                    r = o + e
                    s = v16[e]
                    G[b, r, pl.ds(0, 16)] = G[b, r, pl.ds(0, 16)] * s
                    G[b, r, pl.ds(16, 16)] = G[b, r, pl.ds(16, 16)] * s
                return 0
            lax.fori_loop(0, C // 16, _sc, 0)

        def scatter(b):
            cps = [pltpu.async_copy(G.at[b, pl.ds(j * SUB, SUB)],
                                    acc.at[rv.at[b, j]], sem_s, add=True)
                   for j in range(NSUB)]
            for cp in cps:
                cp.wait()

        issue_in(0, 0)
        wait_in(0)
        issue_gather(0)
        issue_in(1, 1)

        def pair(i2, _):
            for b in range(2):
                k = i2 * 2 + b
                wait_gather(b)
                wait_in(1 - b)
                issue_gather(1 - b)
                scale(b)
                scatter(b)

                @pl.when(k < NCH - 2)
                def _():
                    issue_in(k + 2, b)
            return 0

        lax.fori_loop(0, NCH // 2, pair, 0)
        wait_gather(0)
        scale(0)
        scatter(0)

        plsc.subcore_barrier()
        pltpu.sync_copy(acc.at[pl.ds(sid * RPT, RPT)],
                        out_hbm.at[core, pl.ds(sid * RPT, RPT)])
        plsc.subcore_barrier()

    run_pass(gin, rin, vin, nin2)
    run_pass(gout, rout, vout, nout2)


@jax.jit
def _spmm2(x2, gin, rin, vin, gout, rout, vout):
    mesh = plsc.VectorSubcoreMesh(core_axis_name="c", subcore_axis_name="s")
    f = pl.kernel(
        _spmm_body,
        out_type=(jax.ShapeDtypeStruct((2, N, HALF), jnp.float32),
                  jax.ShapeDtypeStruct((2, N, HALF), jnp.float32)),
        mesh=mesh,
        scratch_types=(
            pltpu.VMEM_SHARED((N, HALF), jnp.float32),
            pltpu.VMEM((2, C), jnp.int32),
            pltpu.VMEM((2, C), jnp.float32),
            pltpu.VMEM((2, NSUB, SUB), jnp.int32),
            pltpu.VMEM((2, C, HALF), jnp.float32),
            pltpu.VMEM((ZR, HALF), jnp.float32),
            pltpu.SemaphoreType.DMA,
            pltpu.SemaphoreType.DMA,
            pltpu.SemaphoreType.DMA,
        ),
        compiler_params=pltpu.CompilerParams(use_tc_tiling_on_sc=False),
    )
    return f(x2, gin, rin, vin, gout, rout, vout)


BN = 2000


def _layer_body(ego_ref, nin_ref, nout_ref, w_ref, ego_o_ref, emb_o_ref):
    ego = ego_ref[...]
    w = w_ref[...]
    acc = jnp.dot(ego, w[0:64], preferred_element_type=jnp.float32)
    acc = acc + jnp.dot(nin_ref[0], w[64:96], preferred_element_type=jnp.float32)
    acc = acc + jnp.dot(nin_ref[1], w[96:128], preferred_element_type=jnp.float32)
    acc = acc + jnp.dot(nout_ref[0], w[128:160], preferred_element_type=jnp.float32)
    acc = acc + jnp.dot(nout_ref[1], w[160:192], preferred_element_type=jnp.float32)
    ego_n = jnp.where(acc >= 0, acc, 0.01 * acc)
    ego_o_ref[...] = ego_n
    nrm = jnp.sqrt(jnp.sum(ego_n * ego_n, axis=1, keepdims=True))
    emb_o_ref[...] = ego_n / jnp.maximum(nrm, 1e-12)


@jax.jit
def _layer(ego, nin2, nout2, w):
    return pl.pallas_call(
        _layer_body,
        grid=(N // BN,),
        in_specs=[
            pl.BlockSpec((BN, D), lambda i: (i, 0)),
            pl.BlockSpec((2, BN, HALF), lambda i: (0, i, 0)),
            pl.BlockSpec((2, BN, HALF), lambda i: (0, i, 0)),
            pl.BlockSpec((3 * D, D), lambda i: (0, 0)),
        ],
        out_specs=[
            pl.BlockSpec((BN, D), lambda i: (i, 0)),
            pl.BlockSpec((BN, D), lambda i: (i, 0)),
        ],
        out_shape=[
            jax.ShapeDtypeStruct((N, D), jnp.float32),
            jax.ShapeDtypeStruct((N, D), jnp.float32),
        ],
    )(ego, nin2, nout2, w)


def _kg_body(attr_ref, e0_ref, e1_ref, e2_ref, out_ref):
    i = pl.program_id(0)
    a = attr_ref[...].reshape(BN, 1)
    lbl = lax.broadcasted_iota(jnp.int32, (BN, R), 1).astype(jnp.float32)
    oh = (a == lbl).astype(jnp.float32)
    dn = (((0,), (0,)), ((), ()))
    r0 = lax.dot_general(oh, e0_ref[...], dn, preferred_element_type=jnp.float32)
    r1 = lax.dot_general(oh, e1_ref[...], dn, preferred_element_type=jnp.float32)
    r2 = lax.dot_general(oh, e2_ref[...], dn, preferred_element_type=jnp.float32)
    contrib = jnp.concatenate([r0, r1, r2], axis=1)

    @pl.when(i == 0)
    def _():
        out_ref[...] = jnp.zeros_like(out_ref)

    out_ref[...] += contrib


@jax.jit
def _kg(attr_f, e0, e1, e2):
    return pl.pallas_call(
        _kg_body,
        grid=(N // BN,),
        in_specs=[
            pl.BlockSpec((1, 1, BN), lambda i: (i, 0, 0)),
            pl.BlockSpec((BN, D), lambda i: (i, 0)),
            pl.BlockSpec((BN, D), lambda i: (i, 0)),
            pl.BlockSpec((BN, D), lambda i: (i, 0)),
        ],
        out_specs=pl.BlockSpec((R, 3 * D), lambda i: (0, 0)),
        out_shape=jax.ShapeDtypeStruct((R, 3 * D), jnp.float32),
    )(attr_f, e0, e1, e2)


def kernel(edge_emb, W1, W2, Hin_idx, Hin_val, Hout_idx, Hout_val, edge_attr):
    rin = Hin_idx[0]
    rout = Hout_idx[0]
    gin = jnp.stack([Hin_idx[1] * 2, Hin_idx[1] * 2 + 1])
    gout = jnp.stack([Hout_idx[1] * 2, Hout_idx[1] * 2 + 1])

    ego0 = edge_emb
    nin2, nout2 = _spmm2(ego0.reshape(2 * N, HALF), gin, rin, Hin_val,
                         gout, rout, Hout_val)
    ego1, emb1 = _layer(ego0, nin2, nout2, W1)
    nin2b, nout2b = _spmm2(ego1.reshape(2 * N, HALF), gin, rin, Hin_val,
                           gout, rout, Hout_val)
    _, emb2 = _layer(ego1, nin2b, nout2b, W2)

    attr_f = edge_attr.astype(jnp.float32).reshape(N // BN, 1, BN)
    return _kg(attr_f, ego0, emb1, emb2)


# speedup vs baseline: 10.3204x; 1.1855x over previous
import functools

import jax
import jax.numpy as jnp
from jax import lax
from jax.experimental import pallas as pl
from jax.experimental.pallas import tpu as pltpu
from jax.experimental.pallas import tpu_sc as plsc

N = 50000
E = 800000
D = 64
R = 10
HALF = 32

NS = 16
SUB = 80
NSUB = 5
C = SUB * NSUB
EPT = E // NS
NCH = EPT // C
RPT = N // NS
ZR = 25


def _spmm_body(x2, gin, rin, vin, gout, rout, vout, nin2, nout2,
               acc, gv, vv, rv, G, zbuf, sem_in, sem_g, sem_s):
    core = lax.axis_index("c")
    sid = lax.axis_index("s")

    def _zb(r, _):
        z16 = jnp.zeros((16,), jnp.float32)
        zbuf[r, pl.ds(0, 16)] = z16
        zbuf[r, pl.ds(16, 16)] = z16
        return 0
    lax.fori_loop(0, ZR, _zb, 0)

    def run_pass(g_hbm, rows_hbm, vals_hbm, out_hbm):
        def _zz(zg, _):
            cps = [pltpu.async_copy(
                zbuf, acc.at[pl.ds(sid * RPT + (zg * 5 + u) * ZR, ZR)], sem_s)
                for u in range(5)]
            for cp in cps:
                cp.wait()
            return 0
        lax.fori_loop(0, RPT // ZR // 5, _zz, 0)
        plsc.subcore_barrier()

        def issue_in(k, p, r):
            ebase = sid * EPT + k * C
            pltpu.async_copy(g_hbm.at[core, pl.ds(ebase, C)], gv.at[p], sem_in)
            pltpu.async_copy(vals_hbm.at[pl.ds(ebase, C)], vv.at[p], sem_in)
            for j in range(NSUB):
                pltpu.async_copy(rows_hbm.at[pl.ds(ebase + j * SUB, SUB)],
                                 rv.at[r, j], sem_in)

        def wait_in(p, r):
            pltpu.make_async_copy(g_hbm.at[core, pl.ds(0, C)],
                                  gv.at[p], sem_in).wait()
            pltpu.make_async_copy(vals_hbm.at[pl.ds(0, C)],
                                  vv.at[p], sem_in).wait()
            for j in range(NSUB):
                pltpu.make_async_copy(rows_hbm.at[pl.ds(0, SUB)],
                                      rv.at[r, j], sem_in).wait()

        def issue_gather(p):
            for j in range(NSUB):
                pltpu.async_copy(x2.at[gv.at[p, pl.ds(j * SUB, SUB)]],
                                 G.at[p, pl.ds(j * SUB, SUB)], sem_g)

        def wait_gather(p):
            for j in range(NSUB):
                pltpu.make_async_copy(
                    x2.at[gv.at[p, pl.ds(j * SUB, SUB)]],
                    G.at[p, pl.ds(j * SUB, SUB)], sem_g).wait()

        def scale(p):
            def _sc(fg, _):
                o = fg * 16
                v16 = vv[p, pl.ds(o, 16)]
                for e in range(16):
                    row = o + e
                    s = v16[e]
                    G[p, row, pl.ds(0, 16)] = G[p, row, pl.ds(0, 16)] * s
                    G[p, row, pl.ds(16, 16)] = G[p, row, pl.ds(16, 16)] * s
                return 0
            lax.fori_loop(0, C // 16, _sc, 0)

        def issue_scatter(p, r):
            for j in range(NSUB):
                pltpu.async_copy(G.at[p, pl.ds(j * SUB, SUB)],
                                 acc.at[rv.at[r, j]], sem_s, add=True)

        def drain_scatter(p, r):
            for j in range(NSUB):
                pltpu.make_async_copy(G.at[p, pl.ds(j * SUB, SUB)],
                                      acc.at[rv.at[r, j]], sem_s).wait()

        issue_in(0, 0, 0)
        wait_in(0, 0)
        issue_gather(0)
        issue_in(1, 1, 1)

        def quad(i4, _):
            for b4 in range(4):
                k = i4 * 4 + b4
                p = b4 % 2
                pn = 1 - p
                rn = (b4 + 1) % 4
                rp = (b4 - 1) % 4
                r2 = (b4 + 2) % 4
                wait_gather(p)

                @pl.when(k < NCH - 1)
                def _():
                    wait_in(pn, rn)

                @pl.when(k > 0)
                def _():
                    drain_scatter(pn, rp)

                @pl.when(k < NCH - 1)
                def _():
                    issue_gather(pn)

                scale(p)
                issue_scatter(p, b4)

                @pl.when(k < NCH - 2)
                def _():
                    issue_in(k + 2, p, r2)
            return 0

        lax.fori_loop(0, NCH // 4, quad, 0)
        wait_gather(0)
        drain_scatter(1, 3)
        scale(0)
        issue_scatter(0, 0)
        drain_scatter(0, 0)

        plsc.subcore_barrier()
        pltpu.sync_copy(acc.at[pl.ds(sid * RPT, RPT)],
                        out_hbm.at[core, pl.ds(sid * RPT, RPT)])
        plsc.subcore_barrier()

    run_pass(gin, rin, vin, nin2)
    run_pass(gout, rout, vout, nout2)


@jax.jit
def _spmm2(x2, gin, rin, vin, gout, rout, vout):
    mesh = plsc.VectorSubcoreMesh(core_axis_name="c", subcore_axis_name="s")
    f = pl.kernel(
        _spmm_body,
        out_type=(jax.ShapeDtypeStruct((2, N, HALF), jnp.float32),
                  jax.ShapeDtypeStruct((2, N, HALF), jnp.float32)),
        mesh=mesh,
        scratch_types=(
            pltpu.VMEM_SHARED((N, HALF), jnp.float32),
            pltpu.VMEM((2, C), jnp.int32),
            pltpu.VMEM((2, C), jnp.float32),
            pltpu.VMEM((4, NSUB, SUB), jnp.int32),
            pltpu.VMEM((2, C, HALF), jnp.float32),
            pltpu.VMEM((ZR, HALF), jnp.float32),
            pltpu.SemaphoreType.DMA,
            pltpu.SemaphoreType.DMA,
            pltpu.SemaphoreType.DMA,
        ),
        compiler_params=pltpu.CompilerParams(use_tc_tiling_on_sc=False),
    )
    return f(x2, gin, rin, vin, gout, rout, vout)


BN = 2000


def _layer_body(ego_ref, nin_ref, nout_ref, w_ref, ego_o_ref, emb_o_ref):
    ego = ego_ref[...]
    w = w_ref[...]
    acc = jnp.dot(ego, w[0:64], preferred_element_type=jnp.float32)
    acc = acc + jnp.dot(nin_ref[0], w[64:96], preferred_element_type=jnp.float32)
    acc = acc + jnp.dot(nin_ref[1], w[96:128], preferred_element_type=jnp.float32)
    acc = acc + jnp.dot(nout_ref[0], w[128:160], preferred_element_type=jnp.float32)
    acc = acc + jnp.dot(nout_ref[1], w[160:192], preferred_element_type=jnp.float32)
    ego_n = jnp.where(acc >= 0, acc, 0.01 * acc)
    ego_o_ref[...] = ego_n
    nrm = jnp.sqrt(jnp.sum(ego_n * ego_n, axis=1, keepdims=True))
    emb_o_ref[...] = ego_n / jnp.maximum(nrm, 1e-12)


@jax.jit
def _layer(ego, nin2, nout2, w):
    return pl.pallas_call(
        _layer_body,
        grid=(N // BN,),
        in_specs=[
            pl.BlockSpec((BN, D), lambda i: (i, 0)),
            pl.BlockSpec((2, BN, HALF), lambda i: (0, i, 0)),
            pl.BlockSpec((2, BN, HALF), lambda i: (0, i, 0)),
            pl.BlockSpec((3 * D, D), lambda i: (0, 0)),
        ],
        out_specs=[
            pl.BlockSpec((BN, D), lambda i: (i, 0)),
            pl.BlockSpec((BN, D), lambda i: (i, 0)),
        ],
        out_shape=[
            jax.ShapeDtypeStruct((N, D), jnp.float32),
            jax.ShapeDtypeStruct((N, D), jnp.float32),
        ],
    )(ego, nin2, nout2, w)


def _kg_body(attr_ref, e0_ref, e1_ref, e2_ref, out_ref):
    i = pl.program_id(0)
    a = attr_ref[...].reshape(BN, 1)
    lbl = lax.broadcasted_iota(jnp.int32, (BN, R), 1).astype(jnp.float32)
    oh = (a == lbl).astype(jnp.float32)
    dn = (((0,), (0,)), ((), ()))
    r0 = lax.dot_general(oh, e0_ref[...], dn, preferred_element_type=jnp.float32)
    r1 = lax.dot_general(oh, e1_ref[...], dn, preferred_element_type=jnp.float32)
    r2 = lax.dot_general(oh, e2_ref[...], dn, preferred_element_type=jnp.float32)
    contrib = jnp.concatenate([r0, r1, r2], axis=1)

    @pl.when(i == 0)
    def _():
        out_ref[...] = jnp.zeros_like(out_ref)

    out_ref[...] += contrib


@jax.jit
def _kg(attr_f, e0, e1, e2):
    return pl.pallas_call(
        _kg_body,
        grid=(N // BN,),
        in_specs=[
            pl.BlockSpec((1, 1, BN), lambda i: (i, 0, 0)),
            pl.BlockSpec((BN, D), lambda i: (i, 0)),
            pl.BlockSpec((BN, D), lambda i: (i, 0)),
            pl.BlockSpec((BN, D), lambda i: (i, 0)),
        ],
        out_specs=pl.BlockSpec((R, 3 * D), lambda i: (0, 0)),
        out_shape=jax.ShapeDtypeStruct((R, 3 * D), jnp.float32),
    )(attr_f, e0, e1, e2)


def kernel(edge_emb, W1, W2, Hin_idx, Hin_val, Hout_idx, Hout_val, edge_attr):
    rin = Hin_idx[0]
    rout = Hout_idx[0]
    gin = jnp.stack([Hin_idx[1] * 2, Hin_idx[1] * 2 + 1])
    gout = jnp.stack([Hout_idx[1] * 2, Hout_idx[1] * 2 + 1])

    ego0 = edge_emb
    nin2, nout2 = _spmm2(ego0.reshape(2 * N, HALF), gin, rin, Hin_val,
                         gout, rout, Hout_val)
    ego1, emb1 = _layer(ego0, nin2, nout2, W1)
    nin2b, nout2b = _spmm2(ego1.reshape(2 * N, HALF), gin, rin, Hin_val,
                           gout, rout, Hout_val)
    _, emb2 = _layer(ego1, nin2b, nout2b, W2)

    attr_f = edge_attr.astype(jnp.float32).reshape(N // BN, 1, BN)
    return _kg(attr_f, ego0, emb1, emb2)
